# baseline (speedup 1.0000x reference)
import jax
import jax.numpy as jnp
from jax.experimental import pallas as pl


def _head_kernel(gx_ref, wl1_ref, bl1_ref, wl2_ref, bl2_ref, out_ref):
    gx = gx_ref[...]
    p = jnp.maximum(gx @ wl1_ref[...] + bl1_ref[...][None, :], 0.0)
    out_ref[...] = p @ wl2_ref[...] + bl2_ref[...][None, :]


def kernel(x, edge_index, batch, w_emb, b_emb, w1, b1, w2, b2, w3, b3,
           w_l1, b_l1, w_l2, b_l2):
    num_nodes = x.shape[0]
    G = 128
    loop = jnp.arange(num_nodes, dtype=edge_index.dtype)
    row = jnp.concatenate([edge_index[0], loop])
    col = jnp.concatenate([edge_index[1], loop])
    ew = jnp.ones((row.shape[0],), jnp.float32)
    h = x @ w_emb + b_emb
    deg = jnp.zeros((num_nodes,), jnp.float32).at[col].add(ew)
    dinv = jnp.where(deg > 0, jax.lax.rsqrt(deg), 0.0)
    norm = dinv[row] * ew * dinv[col]
    for W, b in ((w1, b1), (w2, b2), (w3, b3)):
        xl = h @ W
        msg = xl[row] * norm[:, None]
        out = jnp.zeros((num_nodes, xl.shape[1]), xl.dtype).at[col].add(msg)
        h = jax.nn.relu(out + b)
    sums = jax.ops.segment_sum(h, batch, num_segments=G)
    cnt = jax.ops.segment_sum(jnp.ones((num_nodes, 1), jnp.float32), batch,
                              num_segments=G)
    graph_x = sums / jnp.maximum(cnt, 1.0)
    return pl.pallas_call(
        _head_kernel,
        out_shape=jax.ShapeDtypeStruct((G, w_l2.shape[1]), jnp.float32),
    )(graph_x, w_l1, b_l1, w_l2, b_l2)


# speedup vs baseline: 13.1832x; 13.1832x over previous
import dataclasses

import jax
import jax.numpy as jnp
from jax import lax
from jax.experimental import pallas as pl
from jax.experimental.pallas import tpu as pltpu
from jax.experimental.pallas import tpu_sc as plsc

_SC_PARAMS = dataclasses.replace(pltpu.CompilerParams(),
                                 needs_layout_passes=False)

_N = 100000
_E = 1600000
_H = 64
_G = 128
_NCORE = 2
_NSUB = 16
_NW = _NCORE * _NSUB
_C = 16
_CN = 6256
_W = 128
_TBL = 6272
_SHIFT = 13
_B = 240
_EB = 2000
_S = _E // _NW
_NBLK_A = _S // _EB
_DEGP = 100096
_STRIPE = _DEGP // _NSUB
_EPAD = _E + _C * _B
_TRASH = _EPAD
_EPAD_T = _EPAD + _B
_R = 1024
_GRID_N = (_N + _R - 1) // _R


def _sc_mesh():
    return plsc.VectorSubcoreMesh(core_axis_name="c", subcore_axis_name="s",
                                  num_cores=_NCORE, num_subcores=_NSUB)


def _a1_body(col_hbm, cnt_hbm, deg_hbm, col_v, ones_v, cnt_v, zb_v, deg_sh):
    core = lax.axis_index("c")
    sid = lax.axis_index("s")
    wid = sid * _NCORE + core
    base = wid * _S
    zf = jnp.zeros((16,), jnp.float32)
    of = jnp.full((16,), 1.0, jnp.float32)

    @pl.loop(0, _STRIPE, step=16)
    def _(i):
        zb_v[pl.ds(i, 16)] = zf

    @pl.loop(0, _EB, step=16)
    def _(i):
        ones_v[pl.ds(i, 16)] = of

    pltpu.sync_copy(zb_v, deg_sh.at[pl.ds(sid * _STRIPE, _STRIPE)])
    plsc.subcore_barrier()

    thr = [jnp.full((16,), t * _CN, jnp.int32) for t in range(1, _C)]

    def blk(b, cums):
        pltpu.sync_copy(col_hbm.at[pl.ds(base + b * _EB, _EB)], col_v)
        pltpu.sync_copy(ones_v, deg_sh.at[col_v], add=True)

        def vec(i, cums):
            v = col_v[pl.ds(i * 16, 16)]
            return tuple(cums[t] + (v >= thr[t]).astype(jnp.int32)
                         for t in range(_C - 1))

        return lax.fori_loop(0, _EB // 16, vec, cums)

    cums = lax.fori_loop(0, _NBLK_A, blk,
                         tuple(jnp.zeros((16,), jnp.int32) for _ in range(_C - 1)))

    hi = jnp.full((16,), _S // 16, jnp.int32)
    seq = list(cums) + [jnp.zeros((16,), jnp.int32)]
    for k in range(_C):
        cnt_v[pl.ds(k * 16, 16)] = hi - seq[k]
        hi = seq[k]
    pltpu.sync_copy(cnt_v, cnt_hbm.at[pl.ds(wid * (_C * 16), _C * 16)])
    plsc.subcore_barrier()
    pltpu.sync_copy(deg_sh.at[pl.ds(sid * _STRIPE, _STRIPE)], zb_v)
    pltpu.sync_copy(zb_v, deg_hbm.at[pl.ds(core * _DEGP + sid * _STRIPE, _STRIPE)])


def _phase_a1(col):
    return pl.kernel(
        _a1_body,
        out_type=(jax.ShapeDtypeStruct((_NW * _C * 16,), jnp.int32),
                  jax.ShapeDtypeStruct((_NCORE * _DEGP,), jnp.float32)),
        mesh=_sc_mesh(),
        scratch_types=[pltpu.VMEM((_EB,), jnp.int32),
                       pltpu.VMEM((_EB,), jnp.float32),
                       pltpu.VMEM((_C * 16,), jnp.int32),
                       pltpu.VMEM((_STRIPE,), jnp.float32),
                       pltpu.VMEM_SHARED((_DEGP,), jnp.float32)],
        name="sc_count_deg",
        compiler_params=_SC_PARAMS,
    )(col)


def _a2_body(row_hbm, col_hbm, ofs_hbm, padpos_hbm, padval_hbm, pk_hbm,
             row_v, col_v, pos_v, pkd_v, ofs_v, pp_v, pv_v):
    core = lax.axis_index("c")
    sid = lax.axis_index("s")
    wid = sid * _NCORE + core
    base = wid * _S
    thr = [jnp.full((16,), t * _CN, jnp.int32) for t in range(1, _C)]
    shv = jnp.full((16,), _SHIFT, jnp.int32)
    cnv = jnp.full((16,), _CN, jnp.int32)
    zi = jnp.zeros((16,), jnp.int32)

    pltpu.sync_copy(ofs_hbm.at[pl.ds(wid * (_C * 16), _C * 16)], ofs_v)
    ofs0 = tuple(ofs_v[pl.ds(k * 16, 16)] for k in range(_C))

    def blk(b, ofs):
        pltpu.sync_copy(row_hbm.at[pl.ds(base + b * _EB, _EB)], row_v)
        pltpu.sync_copy(col_hbm.at[pl.ds(base + b * _EB, _EB)], col_v)

        def vec(i, ofs):
            cv = col_v[pl.ds(i * 16, 16)]
            rv = row_v[pl.ds(i * 16, 16)]
            c = zi
            for t in range(_C - 1):
                c = c + (cv >= thr[t]).astype(jnp.int32)
            localv = cv - c * cnv
            pkd = lax.shift_left(rv, shv) | localv
            pos = zi
            out = []
            for k in range(_C):
                mk = c == k
                pos = jnp.where(mk, ofs[k], pos)
                out.append(ofs[k] + mk.astype(jnp.int32))
            pos_v[pl.ds(i * 16, 16)] = pos
            pkd_v[pl.ds(i * 16, 16)] = pkd
            return tuple(out)

        ofs = lax.fori_loop(0, _EB // 16, vec, ofs)
        pltpu.sync_copy(pkd_v, pk_hbm.at[pos_v])
        return ofs

    lax.fori_loop(0, _NBLK_A, blk, ofs0)

    @pl.when(wid < _C)
    def _():
        pltpu.sync_copy(padpos_hbm.at[pl.ds(wid * _B, _B)], pp_v)
        pltpu.sync_copy(padval_hbm.at[pl.ds(wid * _B, _B)], pv_v)
        pltpu.sync_copy(pv_v, pk_hbm.at[pp_v])


def _phase_a2(row, col, ofs, pad_pos, pad_val):
    return pl.kernel(
        _a2_body,
        out_type=jax.ShapeDtypeStruct((_EPAD_T,), jnp.int32),
        mesh=_sc_mesh(),
        scratch_types=[pltpu.VMEM((_EB,), jnp.int32),
                       pltpu.VMEM((_EB,), jnp.int32),
                       pltpu.VMEM((_EB,), jnp.int32),
                       pltpu.VMEM((_EB,), jnp.int32),
                       pltpu.VMEM((_C * 16,), jnp.int32),
                       pltpu.VMEM((_B,), jnp.int32),
                       pltpu.VMEM((_B,), jnp.int32)],
        name="sc_partition",
        compiler_params=_SC_PARAMS,
    )(row, col, ofs, pad_pos, pad_val)


def _c_body(y_hbm, pk_hbm, meta_hbm, acc_hbm,
            pk_v, gi_v, li_v, rows_v, gi2_v, li2_v, rows2_v,
            zv, wb_v, meta_v, table, sem_a, sem_b):
    core = lax.axis_index("c")
    sid = lax.axis_index("s")
    lane = lax.iota(jnp.int32, 16)
    zf = jnp.zeros((16,), jnp.float32)
    zi = jnp.zeros((16,), jnp.int32)
    shv = jnp.full((16,), _SHIFT, jnp.int32)
    mkv = jnp.full((16,), (1 << _SHIFT) - 1, jnp.int32)

    pltpu.sync_copy(meta_hbm, meta_v)

    @pl.loop(0, 64)
    def _(r):
        @pl.loop(0, _W, step=16)
        def _(q):
            zv[r, pl.ds(q, 16)] = zf

    sv = meta_v[pl.ds(0, 16)]
    nv = meta_v[pl.ds(16, 16)]

    bufs = ((gi_v, li_v, rows_v, sem_a), (gi2_v, li2_v, rows2_v, sem_b))

    for kk in range(_C // _NCORE):
        kch = core * (_C // _NCORE) + kk
        mk = lane == kch
        st = pl.multiple_of(jnp.sum(jnp.where(mk, sv, zi)), 16)
        nb = jnp.sum(jnp.where(mk, nv, zi))

        for t in range(7):
            j = sid + t * _NSUB

            @pl.when(j < _TBL // 64)
            def _():
                pltpu.sync_copy(zv, table.at[pl.ds(j * 64, 64), :])

        plsc.subcore_barrier()

        b0 = (nb * sid) // _NSUB
        b1 = (nb * (sid + 1)) // _NSUB

        def load_start(bi, gi_ref, li_ref, rows_ref, sem):
            off = pl.multiple_of(st + bi * _B, 16)
            pltpu.sync_copy(pk_hbm.at[pl.ds(off, _B)], pk_v)

            def vec(i, _):
                v = pk_v[pl.ds(i * 16, 16)]
                gi_ref[pl.ds(i * 16, 16)] = lax.shift_right_logical(v, shv)
                li_ref[pl.ds(i * 16, 16)] = v & mkv
                return 0

            lax.fori_loop(0, _B // 16, vec, 0)
            pltpu.async_copy(y_hbm.at[gi_ref], rows_ref, sem)

        def drain_scatter(gi_ref, li_ref, rows_ref, sem):
            pltpu.make_async_copy(y_hbm.at[gi_ref], rows_ref, sem).wait()
            pltpu.sync_copy(rows_ref, table.at[li_ref], add=True)

        @pl.when(b0 < b1)
        def _():
            load_start(b0, *bufs[0])

            def pair(j, _):
                for p in (0, 1):
                    bi = b0 + 2 * j + p

                    @pl.when(bi < b1)
                    def _():
                        @pl.when(bi + 1 < b1)
                        def _():
                            load_start(bi + 1, *bufs[1 - p])

                        drain_scatter(*bufs[p])
                return 0

            lax.fori_loop(0, (b1 - b0 + 1) // 2, pair, 0)

        plsc.subcore_barrier()

        wbase = kch * _CN

        def copy_rows(r, nr):
            pltpu.sync_copy(table.at[pl.ds(r, nr), :], wb_v.at[pl.ds(0, nr), :])
            pltpu.sync_copy(wb_v.at[pl.ds(0, nr), :],
                            acc_hbm.at[pl.ds(wbase + r, nr), :])

        nfull = jnp.where(kch < _C - 1, 97, 96)
        for t in range(7):
            j = sid + t * _NSUB

            @pl.when(j < nfull)
            def _():
                copy_rows(j * 64, 64)

        @pl.when((sid == 0) & (kch < _C - 1))
        def _():
            copy_rows(97 * 64, 48)

        @pl.when((sid == 0) & (kch == _C - 1))
        def _():
            copy_rows(96 * 64, 16)

        plsc.subcore_barrier()


def _phase_c(y, packed, meta):
    return pl.kernel(
        _c_body,
        out_type=jax.ShapeDtypeStruct((_N, _W), jnp.float32),
        mesh=_sc_mesh(),
        scratch_types=[pltpu.VMEM((_B,), jnp.int32),
                       pltpu.VMEM((_B,), jnp.int32),
                       pltpu.VMEM((_B,), jnp.int32),
                       pltpu.VMEM((_B, _W), jnp.float32),
                       pltpu.VMEM((_B,), jnp.int32),
                       pltpu.VMEM((_B,), jnp.int32),
                       pltpu.VMEM((_B, _W), jnp.float32),
                       pltpu.VMEM((64, _W), jnp.float32),
                       pltpu.VMEM((64, _W), jnp.float32),
                       pltpu.VMEM((32,), jnp.int32),
                       pltpu.VMEM_SHARED((_TBL, _W), jnp.float32),
                       pltpu.SemaphoreType.DMA,
                       pltpu.SemaphoreType.DMA],
        name="sc_gather_scatter",
        compiler_params=_SC_PARAMS,
    )(y, packed, meta)


def _tck1_body(x_ref, deg_ref, wemb_ref, bemb_ref, w1_ref, dinv_ref, y_ref):
    d = deg_ref[0] + deg_ref[1] + 1.0
    dinv = lax.rsqrt(d)
    h = jnp.dot(x_ref[...], wemb_ref[...],
                preferred_element_type=jnp.float32) + bemb_ref[...][None, :]
    y64 = dinv * jnp.dot(h, w1_ref[...], preferred_element_type=jnp.float32)
    y_ref[...] = jnp.concatenate([y64, jnp.zeros_like(y64)], axis=1)
    dinv_ref[...] = dinv


def _tck1(x, degr, w_emb, b_emb, w1):
    return pl.pallas_call(
        _tck1_body,
        grid=(_GRID_N,),
        in_specs=[pl.BlockSpec((_R, 4), lambda i: (i, 0)),
                  pl.BlockSpec((_NCORE, _R, 1), lambda i: (0, i, 0)),
                  pl.BlockSpec((4, _H), lambda i: (0, 0)),
                  pl.BlockSpec((_H,), lambda i: (0,)),
                  pl.BlockSpec((_H, _H), lambda i: (0, 0))],
        out_specs=[pl.BlockSpec((_R, 1), lambda i: (i, 0)),
                   pl.BlockSpec((_R, _W), lambda i: (i, 0))],
        out_shape=[jax.ShapeDtypeStruct((_DEGP, 1), jnp.float32),
                   jax.ShapeDtypeStruct((_N, _W), jnp.float32)],
        name="tc_embed_y1",
    )(x, degr, w_emb, b_emb, w1)


def _tck2_body(acc_ref, y_ref, dinv_ref, b_ref, w_ref, out_ref):
    dinv = dinv_ref[...]
    a = acc_ref[...][:, :_H] + y_ref[...][:, :_H]
    h = jnp.maximum(dinv * a + b_ref[...][None, :], 0.0)
    o64 = dinv * jnp.dot(h, w_ref[...], preferred_element_type=jnp.float32)
    out_ref[...] = jnp.concatenate([o64, jnp.zeros_like(o64)], axis=1)


def _tck2(acc, y, dinv, b, w):
    return pl.pallas_call(
        _tck2_body,
        grid=(_GRID_N,),
        in_specs=[pl.BlockSpec((_R, _W), lambda i: (i, 0)),
                  pl.BlockSpec((_R, _W), lambda i: (i, 0)),
                  pl.BlockSpec((_R, 1), lambda i: (i, 0)),
                  pl.BlockSpec((_H,), lambda i: (0,)),
                  pl.BlockSpec((_H, _H), lambda i: (0, 0))],
        out_specs=pl.BlockSpec((_R, _W), lambda i: (i, 0)),
        out_shape=jax.ShapeDtypeStruct((_N, _W), jnp.float32),
        name="tc_layer_y",
    )(acc, y, dinv, b, w)


def _tck4_body(acc_ref, y_ref, dinv_ref, b3_ref, batch_ref,
               wl1_ref, bl1_ref, wl2_ref, bl2_ref, out_ref,
               sums_ref, cnt_ref):
    i = pl.program_id(0)

    @pl.when(i == 0)
    def _():
        sums_ref[...] = jnp.zeros_like(sums_ref)
        cnt_ref[...] = jnp.zeros_like(cnt_ref)

    dinv = dinv_ref[...]
    a = acc_ref[...][:, :_H] + y_ref[...][:, :_H]
    h = jnp.maximum(dinv * a + b3_ref[...][None, :], 0.0)
    rows = i * _R + lax.broadcasted_iota(jnp.int32, (_R, 1), 0)
    valid = rows < _N
    h = jnp.where(valid, h, 0.0)
    onehot = jnp.where(
        valid & (batch_ref[...] == lax.broadcasted_iota(jnp.int32, (1, _G), 1)),
        1.0, 0.0)
    sums_ref[...] += lax.dot_general(onehot, h, (((0,), (0,)), ((), ())),
                                     preferred_element_type=jnp.float32)
    cnt_ref[...] += lax.dot_general(onehot, jnp.ones((_R, 1), jnp.float32),
                                    (((0,), (0,)), ((), ())),
                                    preferred_element_type=jnp.float32)

    @pl.when(i == _GRID_N - 1)
    def _():
        gx = sums_ref[...] / jnp.maximum(cnt_ref[...], 1.0)
        p = jnp.maximum(jnp.dot(gx, wl1_ref[...],
                                preferred_element_type=jnp.float32)
                        + bl1_ref[...][None, :], 0.0)
        out_ref[...] = jnp.dot(p, wl2_ref[...],
                               preferred_element_type=jnp.float32) \
            + bl2_ref[...][None, :]


def _tck4(acc, y, dinv, b3, batchT, w_l1, b_l1, w_l2, b_l2):
    return pl.pallas_call(
        _tck4_body,
        grid=(_GRID_N,),
        in_specs=[pl.BlockSpec((_R, _W), lambda i: (i, 0)),
                  pl.BlockSpec((_R, _W), lambda i: (i, 0)),
                  pl.BlockSpec((_R, 1), lambda i: (i, 0)),
                  pl.BlockSpec((_H,), lambda i: (0,)),
                  pl.BlockSpec((_R, 1), lambda i: (i, 0)),
                  pl.BlockSpec((_H, _H), lambda i: (0, 0)),
                  pl.BlockSpec((_H,), lambda i: (0,)),
                  pl.BlockSpec((_H, 3), lambda i: (0, 0)),
                  pl.BlockSpec((3,), lambda i: (0,))],
        out_specs=pl.BlockSpec((_G, 3), lambda i: (0, 0)),
        out_shape=jax.ShapeDtypeStruct((_G, 3), jnp.float32),
        scratch_shapes=[pltpu.VMEM((_G, _H), jnp.float32),
                        pltpu.VMEM((_G, 1), jnp.float32)],
        name="tc_pool_head",
    )(acc, y, dinv, b3, batchT, w_l1, b_l1, w_l2, b_l2)


def kernel(x, edge_index, batch, w_emb, b_emb, w1, b1, w2, b2, w3, b3,
           w_l1, b_l1, w_l2, b_l2):
    row = edge_index[0]
    col = edge_index[1]

    cnt_raw, deg2 = _phase_a1(col)
    cnt = cnt_raw.reshape(_NW, _C, 16)
    lk = jnp.sum(cnt, axis=(0, 2))
    nblk = (lk + (_B - 1)) // _B
    padded = nblk * _B
    start = jnp.cumsum(padded) - padded
    flat = cnt.transpose(1, 0, 2).reshape(_C, _NW * 16)
    excl = jnp.cumsum(flat, axis=1) - flat
    owkl = start[:, None] + excl
    ofs = owkl.reshape(_C, _NW, 16).transpose(1, 0, 2).reshape(-1).astype(jnp.int32)
    npad = padded - lk
    j = jnp.arange(_B, dtype=jnp.int32)[None, :]
    pad_pos = jnp.where(j < npad[:, None], (start + lk)[:, None] + j,
                        _TRASH + j).astype(jnp.int32)
    pad_rows = ((jnp.arange(_C * _B, dtype=jnp.int32) * 97) % _N).reshape(_C, _B)
    pad_loc = _CN + (jnp.arange(_B, dtype=jnp.int32) % (_TBL - _CN))[None, :]
    pad_val = ((pad_rows << _SHIFT) | pad_loc).astype(jnp.int32)

    packed = _phase_a2(row, col, ofs, pad_pos.reshape(-1), pad_val.reshape(-1))
    meta = (jnp.zeros((2, 16), jnp.int32)
            .at[0, :_C].set(start.astype(jnp.int32))
            .at[1, :_C].set(nblk.astype(jnp.int32))).reshape(-1)

    degr = deg2.reshape(_NCORE, _DEGP, 1)
    dinv, y1 = _tck1(x, degr, w_emb, b_emb, w1)
    acc1 = _phase_c(y1, packed, meta)
    y2 = _tck2(acc1, y1, dinv, b1, w2)
    acc2 = _phase_c(y2, packed, meta)
    y3 = _tck2(acc2, y2, dinv, b2, w3)
    acc3 = _phase_c(y3, packed, meta)
    batchT = batch.reshape(_N, 1)
    return _tck4(acc3, y3, dinv, b3, batchT, w_l1, b_l1, w_l2, b_l2)


# speedup vs baseline: 13.5657x; 1.0290x over previous
import dataclasses

import jax
import jax.numpy as jnp
from jax import lax
from jax.experimental import pallas as pl
from jax.experimental.pallas import tpu as pltpu
from jax.experimental.pallas import tpu_sc as plsc

_SC_PARAMS = dataclasses.replace(pltpu.CompilerParams(),
                                 needs_layout_passes=False)

_N = 100000
_E = 1600000
_H = 64
_G = 128
_NCORE = 2
_NSUB = 16
_NW = _NCORE * _NSUB
_C = 16
_CN = 6256
_W = 128
_TBL = 6272
_SHIFT = 13
_B = 240
_EB = 2000
_S = _E // _NW
_NBLK_A = _S // _EB
_DEGP = 100096
_STRIPE = _DEGP // _NSUB
_EPAD = _E + _C * _B
_TRASH = _EPAD
_EPAD_T = _EPAD + _B
_R = 1024
_GRID_N = (_N + _R - 1) // _R


def _sc_mesh():
    return plsc.VectorSubcoreMesh(core_axis_name="c", subcore_axis_name="s",
                                  num_cores=_NCORE, num_subcores=_NSUB)


def _a1_body(col_hbm, cnt_hbm, deg_hbm, col_v, ones_v, cnt_v, zb_v, deg_sh):
    core = lax.axis_index("c")
    sid = lax.axis_index("s")
    wid = sid * _NCORE + core
    base = wid * _S
    zf = jnp.zeros((16,), jnp.float32)
    of = jnp.full((16,), 1.0, jnp.float32)

    @pl.loop(0, _STRIPE, step=16)
    def _(i):
        zb_v[pl.ds(i, 16)] = zf

    @pl.loop(0, _EB, step=16)
    def _(i):
        ones_v[pl.ds(i, 16)] = of

    pltpu.sync_copy(zb_v, deg_sh.at[pl.ds(sid * _STRIPE, _STRIPE)])
    plsc.subcore_barrier()

    thr = [jnp.full((16,), t * _CN, jnp.int32) for t in range(1, _C)]

    def blk(b, cums):
        pltpu.sync_copy(col_hbm.at[pl.ds(base + b * _EB, _EB)], col_v)
        pltpu.sync_copy(ones_v, deg_sh.at[col_v], add=True)

        def vec(i, cums):
            v = col_v[pl.ds(i * 16, 16)]
            return tuple(cums[t] + (v >= thr[t]).astype(jnp.int32)
                         for t in range(_C - 1))

        return lax.fori_loop(0, _EB // 16, vec, cums)

    cums = lax.fori_loop(0, _NBLK_A, blk,
                         tuple(jnp.zeros((16,), jnp.int32) for _ in range(_C - 1)))

    hi = jnp.full((16,), _S // 16, jnp.int32)
    seq = list(cums) + [jnp.zeros((16,), jnp.int32)]
    for k in range(_C):
        cnt_v[pl.ds(k * 16, 16)] = hi - seq[k]
        hi = seq[k]
    pltpu.sync_copy(cnt_v, cnt_hbm.at[pl.ds(wid * (_C * 16), _C * 16)])
    plsc.subcore_barrier()
    pltpu.sync_copy(deg_sh.at[pl.ds(sid * _STRIPE, _STRIPE)], zb_v)
    pltpu.sync_copy(zb_v, deg_hbm.at[pl.ds(core * _DEGP + sid * _STRIPE, _STRIPE)])


def _phase_a1(col):
    return pl.kernel(
        _a1_body,
        out_type=(jax.ShapeDtypeStruct((_NW * _C * 16,), jnp.int32),
                  jax.ShapeDtypeStruct((_NCORE * _DEGP,), jnp.float32)),
        mesh=_sc_mesh(),
        scratch_types=[pltpu.VMEM((_EB,), jnp.int32),
                       pltpu.VMEM((_EB,), jnp.float32),
                       pltpu.VMEM((_C * 16,), jnp.int32),
                       pltpu.VMEM((_STRIPE,), jnp.float32),
                       pltpu.VMEM_SHARED((_DEGP,), jnp.float32)],
        name="sc_count_deg",
        compiler_params=_SC_PARAMS,
    )(col)


def _a2_body(row_hbm, col_hbm, ofs_hbm, padpos_hbm, padval_hbm, pk_hbm,
             row_v, col_v, pos_v, pkd_v, ofs_v, pp_v, pv_v):
    core = lax.axis_index("c")
    sid = lax.axis_index("s")
    wid = sid * _NCORE + core
    base = wid * _S
    thr = [jnp.full((16,), t * _CN, jnp.int32) for t in range(1, _C)]
    shv = jnp.full((16,), _SHIFT, jnp.int32)
    cnv = jnp.full((16,), _CN, jnp.int32)
    zi = jnp.zeros((16,), jnp.int32)

    pltpu.sync_copy(ofs_hbm.at[pl.ds(wid * (_C * 16), _C * 16)], ofs_v)
    lane = lax.iota(jnp.int32, 16)
    sixteen = jnp.full((16,), 16, jnp.int32)
    onei = jnp.full((16,), 1, jnp.int32)

    def blk(b, _):
        pltpu.sync_copy(row_hbm.at[pl.ds(base + b * _EB, _EB)], row_v)
        pltpu.sync_copy(col_hbm.at[pl.ds(base + b * _EB, _EB)], col_v)

        def vec(i, _):
            cv = col_v[pl.ds(i * 16, 16)]
            rv = row_v[pl.ds(i * 16, 16)]
            c = zi
            cw = cv
            for t in range(_C - 1):
                cw = cw - cnv
                c = c + (cw >= zi).astype(jnp.int32)
            localv = cv - c * cnv
            pkd = lax.shift_left(rv, shv) | localv
            idx16 = c * sixteen + lane
            cur = plsc.load_gather(ofs_v, [idx16])
            plsc.store_scatter(ofs_v, [idx16], cur + onei)
            pos_v[pl.ds(i * 16, 16)] = cur
            pkd_v[pl.ds(i * 16, 16)] = pkd
            return 0

        lax.fori_loop(0, _EB // 16, vec, 0)
        pltpu.sync_copy(pkd_v, pk_hbm.at[pos_v])
        return 0

    lax.fori_loop(0, _NBLK_A, blk, 0)

    @pl.when(wid < _C)
    def _():
        pltpu.sync_copy(padpos_hbm.at[pl.ds(wid * _B, _B)], pp_v)
        pltpu.sync_copy(padval_hbm.at[pl.ds(wid * _B, _B)], pv_v)
        pltpu.sync_copy(pv_v, pk_hbm.at[pp_v])


def _phase_a2(row, col, ofs, pad_pos, pad_val):
    return pl.kernel(
        _a2_body,
        out_type=jax.ShapeDtypeStruct((_EPAD_T,), jnp.int32),
        mesh=_sc_mesh(),
        scratch_types=[pltpu.VMEM((_EB,), jnp.int32),
                       pltpu.VMEM((_EB,), jnp.int32),
                       pltpu.VMEM((_EB,), jnp.int32),
                       pltpu.VMEM((_EB,), jnp.int32),
                       pltpu.VMEM((_C * 16,), jnp.int32),
                       pltpu.VMEM((_B,), jnp.int32),
                       pltpu.VMEM((_B,), jnp.int32)],
        name="sc_partition",
        compiler_params=_SC_PARAMS,
    )(row, col, ofs, pad_pos, pad_val)


def _c_body(y_hbm, pk_hbm, meta_hbm, acc_hbm,
            pk_v, gi_v, li_v, rows_v, pk2_v, gi2_v, li2_v, rows2_v,
            zv, wb_v, meta_v, table,
            gsem_a, gsem_b, ssem_a, ssem_b, psem_a, psem_b):
    core = lax.axis_index("c")
    sid = lax.axis_index("s")
    lane = lax.iota(jnp.int32, 16)
    zf = jnp.zeros((16,), jnp.float32)
    zi = jnp.zeros((16,), jnp.int32)
    shv = jnp.full((16,), _SHIFT, jnp.int32)
    mkv = jnp.full((16,), (1 << _SHIFT) - 1, jnp.int32)

    pltpu.sync_copy(meta_hbm, meta_v)

    @pl.loop(0, 64)
    def _(r):
        @pl.loop(0, _W, step=16)
        def _(q):
            zv[r, pl.ds(q, 16)] = zf

    sv = meta_v[pl.ds(0, 16)]
    nv = meta_v[pl.ds(16, 16)]

    bufs = ((pk_v, gi_v, li_v, rows_v, gsem_a, ssem_a, psem_a),
            (pk2_v, gi2_v, li2_v, rows2_v, gsem_b, ssem_b, psem_b))

    for kk in range(_C // _NCORE):
        kch = core * (_C // _NCORE) + kk
        mk = lane == kch
        st = pl.multiple_of(jnp.sum(jnp.where(mk, sv, zi)), 16)
        nb = jnp.sum(jnp.where(mk, nv, zi))

        for t in range(7):
            j = sid + t * _NSUB

            @pl.when(j < _TBL // 64)
            def _():
                pltpu.sync_copy(zv, table.at[pl.ds(j * 64, 64), :])

        plsc.subcore_barrier()

        b0 = (nb * sid) // _NSUB
        b1 = (nb * (sid + 1)) // _NSUB

        def unpack_gather(pkr, gi_ref, li_ref, rows_ref, gsem):
            def vec(i, _):
                v = pkr[pl.ds(i * 16, 16)]
                gi_ref[pl.ds(i * 16, 16)] = lax.shift_right_logical(v, shv)
                li_ref[pl.ds(i * 16, 16)] = v & mkv
                return 0

            lax.fori_loop(0, _B // 16, vec, 0)
            pltpu.async_copy(y_hbm.at[gi_ref], rows_ref, gsem)

        def pk_load(bi, pkr, psem):
            off = pl.multiple_of(st + bi * _B, 16)
            pltpu.async_copy(pk_hbm.at[pl.ds(off, _B)], pkr, psem)

        @pl.when(b0 < b1)
        def _():
            pk_load(b0, bufs[0][0], bufs[0][6])
            pltpu.make_async_copy(pk_hbm.at[pl.ds(0, _B)], bufs[0][0],
                                  bufs[0][6]).wait()
            unpack_gather(bufs[0][0], bufs[0][1], bufs[0][2], bufs[0][3],
                          bufs[0][4])

            @pl.when(b0 + 1 < b1)
            def _():
                pk_load(b0 + 1, bufs[1][0], bufs[1][6])

            def pair(j, _):
                for p in (0, 1):
                    bi = b0 + 2 * j + p
                    cur = bufs[p]
                    nxt = bufs[1 - p]

                    @pl.when(bi < b1)
                    def _():
                        @pl.when(bi + 1 < b1)
                        def _():
                            pltpu.make_async_copy(
                                pk_hbm.at[pl.ds(0, _B)], nxt[0], nxt[6]).wait()

                            @pl.when(bi > b0)
                            def _():
                                pltpu.make_async_copy(
                                    nxt[3], table.at[nxt[2]], nxt[5]).wait()

                            unpack_gather(nxt[0], nxt[1], nxt[2], nxt[3],
                                          nxt[4])

                            @pl.when(bi + 2 < b1)
                            def _():
                                pk_load(bi + 2, cur[0], cur[6])

                        pltpu.make_async_copy(y_hbm.at[cur[1]], cur[3],
                                              cur[4]).wait()
                        pltpu.async_copy(cur[3], table.at[cur[2]], cur[5],
                                         add=True)
                return 0

            lax.fori_loop(0, (b1 - b0 + 1) // 2, pair, 0)

            @pl.when(b1 - b0 > 0)
            def _():
                pltpu.make_async_copy(bufs[0][3], table.at[bufs[0][2]],
                                      bufs[0][5]).wait()

            @pl.when(b1 - b0 > 1)
            def _():
                pltpu.make_async_copy(bufs[1][3], table.at[bufs[1][2]],
                                      bufs[1][5]).wait()

        plsc.subcore_barrier()

        wbase = kch * _CN

        def copy_rows(r, nr):
            pltpu.sync_copy(table.at[pl.ds(r, nr), :], wb_v.at[pl.ds(0, nr), :])
            pltpu.sync_copy(wb_v.at[pl.ds(0, nr), :],
                            acc_hbm.at[pl.ds(wbase + r, nr), :])

        nfull = jnp.where(kch < _C - 1, 97, 96)
        for t in range(7):
            j = sid + t * _NSUB

            @pl.when(j < nfull)
            def _():
                copy_rows(j * 64, 64)

        @pl.when((sid == 0) & (kch < _C - 1))
        def _():
            copy_rows(97 * 64, 48)

        @pl.when((sid == 0) & (kch == _C - 1))
        def _():
            copy_rows(96 * 64, 16)

        plsc.subcore_barrier()


def _phase_c(y, packed, meta):
    return pl.kernel(
        _c_body,
        out_type=jax.ShapeDtypeStruct((_N, _W), jnp.float32),
        mesh=_sc_mesh(),
        scratch_types=[pltpu.VMEM((_B,), jnp.int32),
                       pltpu.VMEM((_B,), jnp.int32),
                       pltpu.VMEM((_B,), jnp.int32),
                       pltpu.VMEM((_B, _W), jnp.float32),
                       pltpu.VMEM((_B,), jnp.int32),
                       pltpu.VMEM((_B,), jnp.int32),
                       pltpu.VMEM((_B,), jnp.int32),
                       pltpu.VMEM((_B, _W), jnp.float32),
                       pltpu.VMEM((64, _W), jnp.float32),
                       pltpu.VMEM((64, _W), jnp.float32),
                       pltpu.VMEM((32,), jnp.int32),
                       pltpu.VMEM_SHARED((_TBL, _W), jnp.float32),
                       pltpu.SemaphoreType.DMA,
                       pltpu.SemaphoreType.DMA,
                       pltpu.SemaphoreType.DMA,
                       pltpu.SemaphoreType.DMA,
                       pltpu.SemaphoreType.DMA,
                       pltpu.SemaphoreType.DMA],
        name="sc_gather_scatter",
        compiler_params=_SC_PARAMS,
    )(y, packed, meta)


def _tck1_body(x_ref, deg_ref, wemb_ref, bemb_ref, w1_ref, dinv_ref, y_ref):
    d = deg_ref[0] + deg_ref[1] + 1.0
    dinv = lax.rsqrt(d)
    h = jnp.dot(x_ref[...], wemb_ref[...],
                preferred_element_type=jnp.float32) + bemb_ref[...][None, :]
    y64 = dinv * jnp.dot(h, w1_ref[...], preferred_element_type=jnp.float32)
    y_ref[...] = jnp.concatenate([y64, jnp.zeros_like(y64)], axis=1)
    dinv_ref[...] = dinv


def _tck1(x, degr, w_emb, b_emb, w1):
    return pl.pallas_call(
        _tck1_body,
        grid=(_GRID_N,),
        in_specs=[pl.BlockSpec((_R, 4), lambda i: (i, 0)),
                  pl.BlockSpec((_NCORE, _R, 1), lambda i: (0, i, 0)),
                  pl.BlockSpec((4, _H), lambda i: (0, 0)),
                  pl.BlockSpec((_H,), lambda i: (0,)),
                  pl.BlockSpec((_H, _H), lambda i: (0, 0))],
        out_specs=[pl.BlockSpec((_R, 1), lambda i: (i, 0)),
                   pl.BlockSpec((_R, _W), lambda i: (i, 0))],
        out_shape=[jax.ShapeDtypeStruct((_DEGP, 1), jnp.float32),
                   jax.ShapeDtypeStruct((_N, _W), jnp.float32)],
        name="tc_embed_y1",
    )(x, degr, w_emb, b_emb, w1)


def _tck2_body(acc_ref, y_ref, dinv_ref, b_ref, w_ref, out_ref):
    dinv = dinv_ref[...]
    a = acc_ref[...][:, :_H] + y_ref[...][:, :_H]
    h = jnp.maximum(dinv * a + b_ref[...][None, :], 0.0)
    o64 = dinv * jnp.dot(h, w_ref[...], preferred_element_type=jnp.float32)
    out_ref[...] = jnp.concatenate([o64, jnp.zeros_like(o64)], axis=1)


def _tck2(acc, y, dinv, b, w):
    return pl.pallas_call(
        _tck2_body,
        grid=(_GRID_N,),
        in_specs=[pl.BlockSpec((_R, _W), lambda i: (i, 0)),
                  pl.BlockSpec((_R, _W), lambda i: (i, 0)),
                  pl.BlockSpec((_R, 1), lambda i: (i, 0)),
                  pl.BlockSpec((_H,), lambda i: (0,)),
                  pl.BlockSpec((_H, _H), lambda i: (0, 0))],
        out_specs=pl.BlockSpec((_R, _W), lambda i: (i, 0)),
        out_shape=jax.ShapeDtypeStruct((_N, _W), jnp.float32),
        name="tc_layer_y",
    )(acc, y, dinv, b, w)


def _tck4_body(acc_ref, y_ref, dinv_ref, b3_ref, batch_ref,
               wl1_ref, bl1_ref, wl2_ref, bl2_ref, out_ref,
               sums_ref, cnt_ref):
    i = pl.program_id(0)

    @pl.when(i == 0)
    def _():
        sums_ref[...] = jnp.zeros_like(sums_ref)
        cnt_ref[...] = jnp.zeros_like(cnt_ref)

    dinv = dinv_ref[...]
    a = acc_ref[...][:, :_H] + y_ref[...][:, :_H]
    h = jnp.maximum(dinv * a + b3_ref[...][None, :], 0.0)
    rows = i * _R + lax.broadcasted_iota(jnp.int32, (_R, 1), 0)
    valid = rows < _N
    h = jnp.where(valid, h, 0.0)
    onehot = jnp.where(
        valid & (batch_ref[...] == lax.broadcasted_iota(jnp.int32, (1, _G), 1)),
        1.0, 0.0)
    sums_ref[...] += lax.dot_general(onehot, h, (((0,), (0,)), ((), ())),
                                     preferred_element_type=jnp.float32)
    cnt_ref[...] += lax.dot_general(onehot, jnp.ones((_R, 1), jnp.float32),
                                    (((0,), (0,)), ((), ())),
                                    preferred_element_type=jnp.float32)

    @pl.when(i == _GRID_N - 1)
    def _():
        gx = sums_ref[...] / jnp.maximum(cnt_ref[...], 1.0)
        p = jnp.maximum(jnp.dot(gx, wl1_ref[...],
                                preferred_element_type=jnp.float32)
                        + bl1_ref[...][None, :], 0.0)
        out_ref[...] = jnp.dot(p, wl2_ref[...],
                               preferred_element_type=jnp.float32) \
            + bl2_ref[...][None, :]


def _tck4(acc, y, dinv, b3, batchT, w_l1, b_l1, w_l2, b_l2):
    return pl.pallas_call(
        _tck4_body,
        grid=(_GRID_N,),
        in_specs=[pl.BlockSpec((_R, _W), lambda i: (i, 0)),
                  pl.BlockSpec((_R, _W), lambda i: (i, 0)),
                  pl.BlockSpec((_R, 1), lambda i: (i, 0)),
                  pl.BlockSpec((_H,), lambda i: (0,)),
                  pl.BlockSpec((_R, 1), lambda i: (i, 0)),
                  pl.BlockSpec((_H, _H), lambda i: (0, 0)),
                  pl.BlockSpec((_H,), lambda i: (0,)),
                  pl.BlockSpec((_H, 3), lambda i: (0, 0)),
                  pl.BlockSpec((3,), lambda i: (0,))],
        out_specs=pl.BlockSpec((_G, 3), lambda i: (0, 0)),
        out_shape=jax.ShapeDtypeStruct((_G, 3), jnp.float32),
        scratch_shapes=[pltpu.VMEM((_G, _H), jnp.float32),
                        pltpu.VMEM((_G, 1), jnp.float32)],
        name="tc_pool_head",
    )(acc, y, dinv, b3, batchT, w_l1, b_l1, w_l2, b_l2)


def kernel(x, edge_index, batch, w_emb, b_emb, w1, b1, w2, b2, w3, b3,
           w_l1, b_l1, w_l2, b_l2):
    row = edge_index[0]
    col = edge_index[1]

    cnt_raw, deg2 = _phase_a1(col)
    cnt = cnt_raw.reshape(_NW, _C, 16)
    lk = jnp.sum(cnt, axis=(0, 2))
    nblk = (lk + (_B - 1)) // _B
    padded = nblk * _B
    start = jnp.cumsum(padded) - padded
    flat = cnt.transpose(1, 0, 2).reshape(_C, _NW * 16)
    excl = jnp.cumsum(flat, axis=1) - flat
    owkl = start[:, None] + excl
    ofs = owkl.reshape(_C, _NW, 16).transpose(1, 0, 2).reshape(-1).astype(jnp.int32)
    npad = padded - lk
    j = jnp.arange(_B, dtype=jnp.int32)[None, :]
    pad_pos = jnp.where(j < npad[:, None], (start + lk)[:, None] + j,
                        _TRASH + j).astype(jnp.int32)
    pad_rows = ((jnp.arange(_C * _B, dtype=jnp.int32) * 97) % _N).reshape(_C, _B)
    pad_loc = _CN + (jnp.arange(_B, dtype=jnp.int32) % (_TBL - _CN))[None, :]
    pad_val = ((pad_rows << _SHIFT) | pad_loc).astype(jnp.int32)

    packed = _phase_a2(row, col, ofs, pad_pos.reshape(-1), pad_val.reshape(-1))
    meta = (jnp.zeros((2, 16), jnp.int32)
            .at[0, :_C].set(start.astype(jnp.int32))
            .at[1, :_C].set(nblk.astype(jnp.int32))).reshape(-1)

    degr = deg2.reshape(_NCORE, _DEGP, 1)
    dinv, y1 = _tck1(x, degr, w_emb, b_emb, w1)
    acc1 = _phase_c(y1, packed, meta)
    y2 = _tck2(acc1, y1, dinv, b1, w2)
    acc2 = _phase_c(y2, packed, meta)
    y3 = _tck2(acc2, y2, dinv, b2, w3)
    acc3 = _phase_c(y3, packed, meta)
    batchT = batch.reshape(_N, 1)
    return _tck4(acc3, y3, dinv, b3, batchT, w_l1, b_l1, w_l2, b_l2)
